# baseline (speedup 1.0000x reference)
import functools

import jax
import jax.numpy as jnp
from jax import lax
from jax.experimental import pallas as pl
from jax.experimental.pallas import tpu as pltpu
from jax.experimental.pallas import tpu_sc as plsc

N = 10000
E = 320000
IN_DIM = 128
HID = 64

NC = 2
NS = 16
NW = NC * NS
B = 80
EPAD = 327680
CHUNKS = EPAD // B
PER_W = CHUNKS // NW
SL = 128
NSTAGE = PER_W // SL
NPAD = 10240
RPT = NPAD // NS
W = 128

R = 2048
G = NPAD // R

f32 = jnp.float32


def _mesh():
    return plsc.VectorSubcoreMesh(core_axis_name="c", subcore_axis_name="s",
                                  num_cores=NC, num_subcores=NS)


@functools.lru_cache(maxsize=None)
def _make_edge_pass(gather=True):
    @functools.partial(
        pl.kernel,
        out_type=jax.ShapeDtypeStruct((NC, NPAD, W), f32),
        mesh=_mesh(),
        scratch_types=[
            pltpu.VMEM((SL, B), jnp.int32),
            pltpu.VMEM((SL, B), jnp.int32),
            pltpu.VMEM((B, W), f32),
            pltpu.VMEM((B, W), f32),
            pltpu.VMEM_SHARED((NPAD, W), f32),
            pltpu.SemaphoreType.DMA,
            pltpu.SemaphoreType.DMA,
        ],
    )
    def edge_pass(table_hbm, src_hbm, dst_hbm, zeros_hbm, out_hbm,
                  src_v, dst_v, rows_a, rows_b, acc_sh, sem_a, sem_b):
        rows_bufs = (rows_a, rows_b)
        gsems = (sem_a, sem_b)
        c = lax.axis_index("c")
        s = lax.axis_index("s")
        wid = s * NC + c
        pltpu.sync_copy(zeros_hbm, acc_sh.at[pl.ds(s * RPT, RPT)])
        if not gather:
            pltpu.sync_copy(table_hbm, rows_a)
        plsc.subcore_barrier()

        def g_start(k, u):
            pltpu.async_copy(table_hbm.at[src_v.at[k]], rows_bufs[u],
                             gsems[u])

        def g_wait(u):
            pltpu.make_async_copy(table_hbm.at[src_v.at[0]], rows_bufs[u],
                                  gsems[u]).wait()

        for h in range(NSTAGE):
            pltpu.sync_copy(src_hbm.at[wid, pl.ds(h * SL, SL)], src_v)
            pltpu.sync_copy(dst_hbm.at[wid, pl.ds(h * SL, SL)], dst_v)
            def step_body(k, carry):
                if gather:
                    g_start(k, 0)
                    g_wait(0)
                pltpu.sync_copy(rows_bufs[0], acc_sh.at[dst_v.at[k]],
                                add=True)
                return carry

            lax.fori_loop(0, SL, step_body, 0)

        plsc.subcore_barrier()
        pltpu.sync_copy(acc_sh.at[pl.ds(s * RPT, RPT)],
                        out_hbm.at[c, pl.ds(s * RPT, RPT)])

    return edge_pass


def _deg_pass(*args):
    return _make_edge_pass(gather=False)(*args)


def _pass128(*args):
    return _make_edge_pass(gather=True)(*args)


def _row_spec(w):
    return pl.BlockSpec((R, w), lambda i: (i, 0))


def _part_spec(w):
    return pl.BlockSpec((NC, R, w), lambda i: (0, i, 0))


def _full_spec(a, b):
    return pl.BlockSpec((a, b), lambda i: (0, 0))


def _t1_body(x_ref, degp_ref, we1_ref, t1_ref):
    x = x_ref[...]
    deg = degp_ref[0, :, 0:1] + degp_ref[1, :, 0:1] + 1.0
    dinv = lax.rsqrt(jnp.maximum(deg, 1.0))
    h1 = jnp.dot(x, we1_ref[...], preferred_element_type=f32)
    dcol = (1.0 - 2.0 * x[:, 0:1]) * dinv
    pad = jnp.zeros((R, 62), f32)
    t1_ref[...] = jnp.concatenate([h1 * dinv, dcol, dinv, pad], axis=1)


def _t1(x, degp, we1):
    return pl.pallas_call(
        _t1_body,
        grid=(G,),
        in_specs=[_row_spec(IN_DIM), _part_spec(W), _full_spec(IN_DIM, HID)],
        out_specs=_row_spec(W),
        out_shape=jax.ShapeDtypeStruct((NPAD, W), f32),
    )(x, degp, we1)


def _t2_body(t1_ref, a1p_ref, we2_ref, we1r0_ref, be1_ref, t2_ref):
    t1 = t1_ref[...]
    dinv = t1[:, 65:66]
    m = dinv * (a1p_ref[0] + a1p_ref[1] + t1)
    pre1 = m[:, :64] + be1_ref[...]
    pre1cf = pre1 + m[:, 64:65] * we1r0_ref[...]
    we2 = we2_ref[...]
    h2 = jnp.dot(jnp.maximum(pre1, 0.0), we2, preferred_element_type=f32)
    h2cf = jnp.dot(jnp.maximum(pre1cf, 0.0), we2, preferred_element_type=f32)
    t2_ref[...] = jnp.concatenate([h2 * dinv, h2cf * dinv], axis=1)


def _t2(t1, a1p, we2, we1r0, be1):
    return pl.pallas_call(
        _t2_body,
        grid=(G,),
        in_specs=[_row_spec(W), _part_spec(W), _full_spec(HID, HID),
                  _full_spec(1, HID), _full_spec(1, HID)],
        out_specs=_row_spec(W),
        out_shape=jax.ShapeDtypeStruct((NPAD, W), f32),
    )(t1, a1p, we2, we1r0, be1)


def _t3_body(t2_ref, a2p_ref, t1_ref, be2_ref, wa11_ref, wa21_ref, ws_ref,
             z_ref, t3a_ref, t3b_ref):
    dinv = t1_ref[:, 65:66]
    m = dinv * (a2p_ref[0] + a2p_ref[1] + t2_ref[...])
    be2 = be2_ref[...]
    z = m[:, :64] + be2
    zcf = m[:, 64:] + be2
    z_s, z_ns, z_s_cf = z[:, :32], z[:, 32:], zcf[:, :32]
    t3a = jnp.concatenate(
        [jnp.dot(z_s, wa11_ref[...], preferred_element_type=f32),
         jnp.dot(z_ns, wa21_ref[...], preferred_element_type=f32)], axis=1)
    t3b = jnp.concatenate(
        [jnp.dot(z_s_cf, wa11_ref[...], preferred_element_type=f32),
         jnp.dot(z_ns, ws_ref[...], preferred_element_type=f32)], axis=1)
    z_ref[...] = z
    t3a_ref[...] = t3a * dinv
    t3b_ref[...] = t3b * dinv


def _t3(t2, a2p, t1, be2, wa11, wa21, ws):
    return pl.pallas_call(
        _t3_body,
        grid=(G,),
        in_specs=[_row_spec(W), _part_spec(W), _row_spec(W),
                  _full_spec(1, HID), _full_spec(32, HID),
                  _full_spec(32, HID), _full_spec(32, HID)],
        out_specs=[_row_spec(64), _row_spec(W), _row_spec(W)],
        out_shape=[jax.ShapeDtypeStruct((NPAD, 64), f32),
                   jax.ShapeDtypeStruct((NPAD, W), f32),
                   jax.ShapeDtypeStruct((NPAD, W), f32)],
    )(t2, a2p, t1, be2, wa11, wa21, ws)


def _t4_body(t3a_ref, t3b_ref, a3ap_ref, a3bp_ref, t1_ref, b3a_ref, b3b_ref,
             wa12_ref, wa22_ref, hs_ref, t4a_ref, t4b_ref, t4c_ref):
    dinv = t1_ref[:, 65:66]
    pa = dinv * (a3ap_ref[0] + a3ap_ref[1] + t3a_ref[...]) + b3a_ref[...]
    pb = dinv * (a3bp_ref[0] + a3bp_ref[1] + t3b_ref[...]) + b3b_ref[...]
    a_s1 = jnp.maximum(pa[:, :64], 0.0)
    a_ns1 = jnp.maximum(pa[:, 64:], 0.0)
    a_scf1 = jnp.maximum(pb[:, :64], 0.0)
    hs_ref[...] = pb[:, 64:]
    t4a_ref[...] = jnp.dot(a_s1, wa12_ref[...],
                           preferred_element_type=f32) * dinv
    t4b_ref[...] = jnp.dot(a_ns1, wa22_ref[...],
                           preferred_element_type=f32) * dinv
    t4c_ref[...] = jnp.dot(a_scf1, wa12_ref[...],
                           preferred_element_type=f32) * dinv


def _t4(t3a, t3b, a3ap, a3bp, t1, b3a, b3b, wa12, wa22):
    return pl.pallas_call(
        _t4_body,
        grid=(G,),
        in_specs=[_row_spec(W), _row_spec(W), _part_spec(W), _part_spec(W),
                  _row_spec(W), _full_spec(1, 128), _full_spec(1, 128),
                  _full_spec(HID, IN_DIM), _full_spec(HID, IN_DIM)],
        out_specs=[_row_spec(64), _row_spec(W), _row_spec(W), _row_spec(W)],
        out_shape=[jax.ShapeDtypeStruct((NPAD, 64), f32),
                   jax.ShapeDtypeStruct((NPAD, W), f32),
                   jax.ShapeDtypeStruct((NPAD, W), f32),
                   jax.ShapeDtypeStruct((NPAD, W), f32)],
    )(t3a, t3b, a3ap, a3bp, t1, b3a, b3b, wa12, wa22)


def _t5_body(t4a_ref, t4b_ref, t4c_ref, a4ap_ref, a4bp_ref, a4cp_ref,
             t1_ref, ba12_ref, ba22_ref, xs_ref, xns_ref, xscf_ref):
    dinv = t1_ref[:, 65:66]
    ba12 = ba12_ref[...]
    xs_ref[...] = dinv * (a4ap_ref[0] + a4ap_ref[1] + t4a_ref[...]) + ba12
    xns_ref[...] = dinv * (a4bp_ref[0] + a4bp_ref[1] + t4b_ref[...]) \
        + ba22_ref[...]
    xscf_ref[...] = dinv * (a4cp_ref[0] + a4cp_ref[1] + t4c_ref[...]) + ba12


def _t5(t4a, t4b, t4c, a4ap, a4bp, a4cp, t1, ba12, ba22):
    return pl.pallas_call(
        _t5_body,
        grid=(G,),
        in_specs=[_row_spec(W), _row_spec(W), _row_spec(W),
                  _part_spec(W), _part_spec(W), _part_spec(W),
                  _row_spec(W), _full_spec(1, 128), _full_spec(1, 128)],
        out_specs=[_row_spec(128), _row_spec(128), _row_spec(128)],
        out_shape=[jax.ShapeDtypeStruct((NPAD, 128), f32),
                   jax.ShapeDtypeStruct((NPAD, 128), f32),
                   jax.ShapeDtypeStruct((NPAD, 128), f32)],
    )(t4a, t4b, t4c, a4ap, a4bp, a4cp, t1, ba12, ba22)


def _t6_body(a_ref, b_ref, out_ref):
    out_ref[...] = lax.dot_general(
        a_ref[...], b_ref[...], (((1,), (1,)), ((), ())),
        preferred_element_type=f32)


def _t6(hs):
    M = 2048
    gm = pl.cdiv(N, M)
    return pl.pallas_call(
        _t6_body,
        grid=(gm, gm),
        in_specs=[pl.BlockSpec((M, HID), lambda i, j: (i, 0)),
                  pl.BlockSpec((M, HID), lambda i, j: (j, 0))],
        out_specs=pl.BlockSpec((M, M), lambda i, j: (i, j)),
        out_shape=jax.ShapeDtypeStruct((N, N), f32),
    )(hs, hs)


def kernel(x, W_e1, b_e1, W_e2, b_e2, Wa11, ba11, Wa12, ba12, Wa21, ba21,
           Wa22, ba22, Ws, bs, edge_index):
    npad_e = EPAD - E
    src = jnp.concatenate(
        [edge_index[0].astype(jnp.int32), jnp.zeros((npad_e,), jnp.int32)]
    ).reshape(NW, PER_W, B)
    pad_dst = N + (jnp.arange(npad_e, dtype=jnp.int32) % (NPAD - N))
    dst = jnp.concatenate(
        [edge_index[1].astype(jnp.int32), pad_dst]
    ).reshape(NW, PER_W, B)
    xp = jnp.pad(x, ((0, NPAD - N), (0, 0)))
    ones = jnp.ones((B, W), f32)
    z128 = jnp.zeros((RPT, W), f32)

    degp = _deg_pass(ones, src, dst, z128)
    t1 = _t1(xp, degp, W_e1)
    a1p = _pass128(t1, src, dst, z128)
    t2 = _t2(t1, a1p, W_e2, W_e1[0:1, :], b_e1.reshape(1, -1))
    a2p = _pass128(t2, src, dst, z128)
    z, t3a, t3b = _t3(t2, a2p, t1, b_e2.reshape(1, -1), Wa11, Wa21, Ws)
    a3ap = _pass128(t3a, src, dst, z128)
    a3bp = _pass128(t3b, src, dst, z128)
    b3a = jnp.concatenate([ba11, ba21]).reshape(1, -1)
    b3b = jnp.concatenate([ba11, bs]).reshape(1, -1)
    hs, t4a, t4b, t4c = _t4(t3a, t3b, a3ap, a3bp, t1, b3a, b3b, Wa12, Wa22)
    a4ap = _pass128(t4a, src, dst, z128)
    a4bp = _pass128(t4b, src, dst, z128)
    a4cp = _pass128(t4c, src, dst, z128)
    s_ = _t6(hs)
    xs, xns, xscf = _t5(t4a, t4b, t4c, a4ap, a4bp, a4cp, t1,
                        ba12.reshape(1, -1), ba22.reshape(1, -1))
    return (z[:N, :32], z[:N, 32:], xs[:N], xns[:N], xscf[:N], s_)


# speedup vs baseline: 3.6310x; 3.6310x over previous
import functools

import jax
import jax.numpy as jnp
from jax import lax
from jax.experimental import pallas as pl
from jax.experimental.pallas import tpu as pltpu
from jax.experimental.pallas import tpu_sc as plsc

N = 10000
E = 320000
IN_DIM = 128
HID = 64

NC = 2
NS = 16
NW = NC * NS
B = 80
CHUNKS = E // B
PER_W = CHUNKS // NW
PW_PAD = 128
NPAD = 10240
RPT = NPAD // NS
W = 128

R = 2048
G = NPAD // R

f32 = jnp.float32


def _mesh():
    return plsc.VectorSubcoreMesh(core_axis_name="c", subcore_axis_name="s",
                                  num_cores=NC, num_subcores=NS)


@functools.lru_cache(maxsize=None)
def _make_edge_pass(gather=True):
    @functools.partial(
        pl.kernel,
        out_type=jax.ShapeDtypeStruct((NC, NPAD, W), f32),
        mesh=_mesh(),
        scratch_types=[
            pltpu.VMEM((PER_W * B,), jnp.int32),
            pltpu.VMEM((PW_PAD, B), jnp.int32),
            pltpu.VMEM((B, W), f32),
            pltpu.VMEM((B, W), f32),
            pltpu.VMEM_SHARED((NPAD, W), f32),
            pltpu.SemaphoreType.DMA,
            pltpu.SemaphoreType.DMA,
        ],
    )
    def edge_pass(table_hbm, src_hbm, dst_hbm, zeros_hbm, out_hbm,
                  src_v, dst_v, rows_a, rows_b, acc_sh, sem_a, sem_b):
        rows_bufs = (rows_a, rows_b)
        gsems = (sem_a, sem_b)
        c = lax.axis_index("c")
        s = lax.axis_index("s")
        wid = s * NC + c
        pltpu.sync_copy(zeros_hbm, acc_sh.at[pl.ds(s * RPT, RPT)])
        pltpu.sync_copy(src_hbm.at[wid], src_v)
        pltpu.sync_copy(dst_hbm.at[wid], dst_v.at[pl.ds(0, PER_W)])
        if not gather:
            pltpu.sync_copy(table_hbm, rows_a)
        plsc.subcore_barrier()

        def g_start(k, u):
            idx = src_v.at[pl.ds(pl.multiple_of(k * B, B), B)]
            pltpu.async_copy(table_hbm.at[idx], rows_bufs[u], gsems[u])

        def g_wait(u):
            pltpu.make_async_copy(table_hbm.at[src_v.at[pl.ds(0, B)]],
                                  rows_bufs[u], gsems[u]).wait()

        def scat(k, ub):
            pltpu.sync_copy(rows_bufs[ub], acc_sh.at[dst_v.at[k]], add=True)

        if gather:
            g_start(0, 0)
            g_start(1, 1)

            def step_body(st, carry):
                for u in range(2):
                    k = 2 * st + u
                    g_wait(u)
                    scat(k, u)
                    if u == 0:
                        g_start(k + 2, u)
                    else:
                        @pl.when(st < (PER_W - 1) // 2 - 1)
                        def _():
                            g_start(k + 2, u)
                return carry

            lax.fori_loop(0, (PER_W - 1) // 2, step_body, 0)
            g_wait(0)
            scat(PER_W - 1, 0)
        else:
            def deg_body(k, carry):
                scat(k, 0)
                return carry

            lax.fori_loop(0, PER_W, deg_body, 0)

        plsc.subcore_barrier()
        pltpu.sync_copy(acc_sh.at[pl.ds(s * RPT, RPT)],
                        out_hbm.at[c, pl.ds(s * RPT, RPT)])

    return edge_pass


def _deg_pass(*args):
    return _make_edge_pass(gather=False)(*args)


def _pass128(*args):
    return _make_edge_pass(gather=True)(*args)


def _row_spec(w):
    return pl.BlockSpec((R, w), lambda i: (i, 0))


def _part_spec(w):
    return pl.BlockSpec((NC, R, w), lambda i: (0, i, 0))


def _full_spec(a, b):
    return pl.BlockSpec((a, b), lambda i: (0, 0))


def _t1_body(x_ref, degp_ref, we1_ref, t1_ref):
    x = x_ref[...]
    deg = degp_ref[0, :, 0:1] + degp_ref[1, :, 0:1] + 1.0
    dinv = lax.rsqrt(jnp.maximum(deg, 1.0))
    h1 = jnp.dot(x, we1_ref[...], preferred_element_type=f32)
    dcol = (1.0 - 2.0 * x[:, 0:1]) * dinv
    pad = jnp.zeros((R, 62), f32)
    t1_ref[...] = jnp.concatenate([h1 * dinv, dcol, dinv, pad], axis=1)


def _t1(x, degp, we1):
    return pl.pallas_call(
        _t1_body,
        grid=(G,),
        in_specs=[_row_spec(IN_DIM), _part_spec(W), _full_spec(IN_DIM, HID)],
        out_specs=_row_spec(W),
        out_shape=jax.ShapeDtypeStruct((NPAD, W), f32),
    )(x, degp, we1)


def _t2_body(t1_ref, a1p_ref, we2_ref, we1r0_ref, be1_ref, t2_ref):
    t1 = t1_ref[...]
    dinv = t1[:, 65:66]
    m = dinv * (a1p_ref[0] + a1p_ref[1] + t1)
    pre1 = m[:, :64] + be1_ref[...]
    pre1cf = pre1 + m[:, 64:65] * we1r0_ref[...]
    we2 = we2_ref[...]
    h2 = jnp.dot(jnp.maximum(pre1, 0.0), we2, preferred_element_type=f32)
    h2cf = jnp.dot(jnp.maximum(pre1cf, 0.0), we2, preferred_element_type=f32)
    t2_ref[...] = jnp.concatenate([h2 * dinv, h2cf * dinv], axis=1)


def _t2(t1, a1p, we2, we1r0, be1):
    return pl.pallas_call(
        _t2_body,
        grid=(G,),
        in_specs=[_row_spec(W), _part_spec(W), _full_spec(HID, HID),
                  _full_spec(1, HID), _full_spec(1, HID)],
        out_specs=_row_spec(W),
        out_shape=jax.ShapeDtypeStruct((NPAD, W), f32),
    )(t1, a1p, we2, we1r0, be1)


def _t3_body(t2_ref, a2p_ref, t1_ref, be2_ref, wa11_ref, wa21_ref, ws_ref,
             z_ref, t3a_ref, t3b_ref):
    dinv = t1_ref[:, 65:66]
    m = dinv * (a2p_ref[0] + a2p_ref[1] + t2_ref[...])
    be2 = be2_ref[...]
    z = m[:, :64] + be2
    zcf = m[:, 64:] + be2
    z_s, z_ns, z_s_cf = z[:, :32], z[:, 32:], zcf[:, :32]
    t3a = jnp.concatenate(
        [jnp.dot(z_s, wa11_ref[...], preferred_element_type=f32),
         jnp.dot(z_ns, wa21_ref[...], preferred_element_type=f32)], axis=1)
    t3b = jnp.concatenate(
        [jnp.dot(z_s_cf, wa11_ref[...], preferred_element_type=f32),
         jnp.dot(z_ns, ws_ref[...], preferred_element_type=f32)], axis=1)
    z_ref[...] = z
    t3a_ref[...] = t3a * dinv
    t3b_ref[...] = t3b * dinv


def _t3(t2, a2p, t1, be2, wa11, wa21, ws):
    return pl.pallas_call(
        _t3_body,
        grid=(G,),
        in_specs=[_row_spec(W), _part_spec(W), _row_spec(W),
                  _full_spec(1, HID), _full_spec(32, HID),
                  _full_spec(32, HID), _full_spec(32, HID)],
        out_specs=[_row_spec(64), _row_spec(W), _row_spec(W)],
        out_shape=[jax.ShapeDtypeStruct((NPAD, 64), f32),
                   jax.ShapeDtypeStruct((NPAD, W), f32),
                   jax.ShapeDtypeStruct((NPAD, W), f32)],
    )(t2, a2p, t1, be2, wa11, wa21, ws)


def _t4_body(t3a_ref, t3b_ref, a3ap_ref, a3bp_ref, t1_ref, b3a_ref, b3b_ref,
             wa12_ref, wa22_ref, hs_ref, t4a_ref, t4b_ref, t4c_ref):
    dinv = t1_ref[:, 65:66]
    pa = dinv * (a3ap_ref[0] + a3ap_ref[1] + t3a_ref[...]) + b3a_ref[...]
    pb = dinv * (a3bp_ref[0] + a3bp_ref[1] + t3b_ref[...]) + b3b_ref[...]
    a_s1 = jnp.maximum(pa[:, :64], 0.0)
    a_ns1 = jnp.maximum(pa[:, 64:], 0.0)
    a_scf1 = jnp.maximum(pb[:, :64], 0.0)
    hs_ref[...] = pb[:, 64:]
    t4a_ref[...] = jnp.dot(a_s1, wa12_ref[...],
                           preferred_element_type=f32) * dinv
    t4b_ref[...] = jnp.dot(a_ns1, wa22_ref[...],
                           preferred_element_type=f32) * dinv
    t4c_ref[...] = jnp.dot(a_scf1, wa12_ref[...],
                           preferred_element_type=f32) * dinv


def _t4(t3a, t3b, a3ap, a3bp, t1, b3a, b3b, wa12, wa22):
    return pl.pallas_call(
        _t4_body,
        grid=(G,),
        in_specs=[_row_spec(W), _row_spec(W), _part_spec(W), _part_spec(W),
                  _row_spec(W), _full_spec(1, 128), _full_spec(1, 128),
                  _full_spec(HID, IN_DIM), _full_spec(HID, IN_DIM)],
        out_specs=[_row_spec(64), _row_spec(W), _row_spec(W), _row_spec(W)],
        out_shape=[jax.ShapeDtypeStruct((NPAD, 64), f32),
                   jax.ShapeDtypeStruct((NPAD, W), f32),
                   jax.ShapeDtypeStruct((NPAD, W), f32),
                   jax.ShapeDtypeStruct((NPAD, W), f32)],
    )(t3a, t3b, a3ap, a3bp, t1, b3a, b3b, wa12, wa22)


def _t5_body(t4a_ref, t4b_ref, t4c_ref, a4ap_ref, a4bp_ref, a4cp_ref,
             t1_ref, ba12_ref, ba22_ref, xs_ref, xns_ref, xscf_ref):
    dinv = t1_ref[:, 65:66]
    ba12 = ba12_ref[...]
    xs_ref[...] = dinv * (a4ap_ref[0] + a4ap_ref[1] + t4a_ref[...]) + ba12
    xns_ref[...] = dinv * (a4bp_ref[0] + a4bp_ref[1] + t4b_ref[...]) \
        + ba22_ref[...]
    xscf_ref[...] = dinv * (a4cp_ref[0] + a4cp_ref[1] + t4c_ref[...]) + ba12


def _t5(t4a, t4b, t4c, a4ap, a4bp, a4cp, t1, ba12, ba22):
    return pl.pallas_call(
        _t5_body,
        grid=(G,),
        in_specs=[_row_spec(W), _row_spec(W), _row_spec(W),
                  _part_spec(W), _part_spec(W), _part_spec(W),
                  _row_spec(W), _full_spec(1, 128), _full_spec(1, 128)],
        out_specs=[_row_spec(128), _row_spec(128), _row_spec(128)],
        out_shape=[jax.ShapeDtypeStruct((NPAD, 128), f32),
                   jax.ShapeDtypeStruct((NPAD, 128), f32),
                   jax.ShapeDtypeStruct((NPAD, 128), f32)],
    )(t4a, t4b, t4c, a4ap, a4bp, a4cp, t1, ba12, ba22)


def _t6_body(a_ref, b_ref, out_ref):
    out_ref[...] = lax.dot_general(
        a_ref[...], b_ref[...], (((1,), (1,)), ((), ())),
        preferred_element_type=f32)


def _t6(hs):
    M = 2048
    gm = pl.cdiv(N, M)
    return pl.pallas_call(
        _t6_body,
        grid=(gm, gm),
        in_specs=[pl.BlockSpec((M, HID), lambda i, j: (i, 0)),
                  pl.BlockSpec((M, HID), lambda i, j: (j, 0))],
        out_specs=pl.BlockSpec((M, M), lambda i, j: (i, j)),
        out_shape=jax.ShapeDtypeStruct((N, N), f32),
    )(hs, hs)


def kernel(x, W_e1, b_e1, W_e2, b_e2, Wa11, ba11, Wa12, ba12, Wa21, ba21,
           Wa22, ba22, Ws, bs, edge_index):
    src = edge_index[0].astype(jnp.int32).reshape(NW, PER_W * B)
    dst = edge_index[1].astype(jnp.int32).reshape(NW, PER_W, B)
    xp = jnp.pad(x, ((0, NPAD - N), (0, 0)))
    ones = jnp.ones((B, W), f32)
    z128 = jnp.zeros((RPT, W), f32)

    degp = _deg_pass(ones, src, dst, z128)
    t1 = _t1(xp, degp, W_e1)
    a1p = _pass128(t1, src, dst, z128)
    t2 = _t2(t1, a1p, W_e2, W_e1[0:1, :], b_e1.reshape(1, -1))
    a2p = _pass128(t2, src, dst, z128)
    z, t3a, t3b = _t3(t2, a2p, t1, b_e2.reshape(1, -1), Wa11, Wa21, Ws)
    a3ap = _pass128(t3a, src, dst, z128)
    a3bp = _pass128(t3b, src, dst, z128)
    b3a = jnp.concatenate([ba11, ba21]).reshape(1, -1)
    b3b = jnp.concatenate([ba11, bs]).reshape(1, -1)
    hs, t4a, t4b, t4c = _t4(t3a, t3b, a3ap, a3bp, t1, b3a, b3b, Wa12, Wa22)
    a4ap = _pass128(t4a, src, dst, z128)
    a4bp = _pass128(t4b, src, dst, z128)
    a4cp = _pass128(t4c, src, dst, z128)
    s_ = _t6(hs)
    xs, xns, xscf = _t5(t4a, t4b, t4c, a4ap, a4bp, a4cp, t1,
                        ba12.reshape(1, -1), ba22.reshape(1, -1))
    return (z[:N, :32], z[:N, 32:], xs[:N], xns[:N], xscf[:N], s_)


# speedup vs baseline: 4.7064x; 1.2961x over previous
import functools

import jax
import jax.numpy as jnp
from jax import lax
from jax.experimental import pallas as pl
from jax.experimental.pallas import tpu as pltpu
from jax.experimental.pallas import tpu_sc as plsc

N = 10000
E = 320000
IN_DIM = 128
HID = 64

NC = 2
NS = 16
NW = NC * NS
B = 80
CHUNKS = E // B
PER_W = CHUNKS // NW
PW_PAD = 128
NPAD = 10240
RPT = NPAD // NS
W = 128

R = 2048
G = NPAD // R

f32 = jnp.float32


def _mesh():
    return plsc.VectorSubcoreMesh(core_axis_name="c", subcore_axis_name="s",
                                  num_cores=NC, num_subcores=NS)


@functools.lru_cache(maxsize=None)
def _make_edge_pass(gather=True):
    @functools.partial(
        pl.kernel,
        out_type=jax.ShapeDtypeStruct((NC, NPAD, W), f32),
        mesh=_mesh(),
        scratch_types=[
            pltpu.VMEM((PER_W * B,), jnp.int32),
            pltpu.VMEM((PW_PAD, B), jnp.int32),
            pltpu.VMEM((B, W), f32),
            pltpu.VMEM((B, W), f32),
            pltpu.VMEM_SHARED((NPAD, W), f32),
            pltpu.SemaphoreType.DMA,
            pltpu.SemaphoreType.DMA,
        ],
    )
    def edge_pass(table_hbm, src_hbm, dst_hbm, zeros_hbm, out_hbm,
                  src_v, dst_v, rows_a, rows_b, acc_sh, sem_a, sem_b):
        rows_bufs = (rows_a, rows_b)
        gsems = (sem_a, sem_b)
        c = lax.axis_index("c")
        s = lax.axis_index("s")
        wid = s * NC + c
        pltpu.sync_copy(zeros_hbm, acc_sh.at[pl.ds(s * RPT, RPT)])
        pltpu.sync_copy(src_hbm.at[wid], src_v)
        pltpu.sync_copy(dst_hbm.at[wid], dst_v.at[pl.ds(0, PER_W)])
        if not gather:
            pltpu.sync_copy(table_hbm, rows_a)
        plsc.subcore_barrier()

        def g_start(k, u):
            idx = src_v.at[pl.ds(pl.multiple_of(k * B, B), B)]
            pltpu.async_copy(table_hbm.at[idx], rows_bufs[u], gsems[u])

        def g_wait(u):
            pltpu.make_async_copy(table_hbm.at[src_v.at[pl.ds(0, B)]],
                                  rows_bufs[u], gsems[u]).wait()

        def scat(k, ub):
            pltpu.sync_copy(rows_bufs[ub], acc_sh.at[dst_v.at[k]], add=True)

        if gather:
            g_start(0, 0)
            g_start(1, 1)

            def step_body(st, carry):
                for u in range(2):
                    k = 2 * st + u
                    g_wait(u)
                    scat(k, u)
                    if u == 0:
                        g_start(k + 2, u)
                    else:
                        @pl.when(st < (PER_W - 1) // 2 - 1)
                        def _():
                            g_start(k + 2, u)
                return carry

            lax.fori_loop(0, (PER_W - 1) // 2, step_body, 0)
            g_wait(0)
            scat(PER_W - 1, 0)
        else:
            def deg_body(k, carry):
                scat(k, 0)
                return carry

            lax.fori_loop(0, PER_W, deg_body, 0)

        plsc.subcore_barrier()
        pltpu.sync_copy(acc_sh.at[pl.ds(s * RPT, RPT)],
                        out_hbm.at[c, pl.ds(s * RPT, RPT)])

    return edge_pass


def _deg_pass(*args):
    return _make_edge_pass(gather=False)(*args)


def _pass128(*args):
    return _make_edge_pass(gather=True)(*args)


def _row_spec(w):
    return pl.BlockSpec((R, w), lambda i: (i, 0))


def _part_spec(w):
    return pl.BlockSpec((NC, R, w), lambda i: (0, i, 0))


def _full_spec(a, b):
    return pl.BlockSpec((a, b), lambda i: (0, 0))


def _t1_body(x_ref, degp_ref, we1_ref, t1_ref):
    x = x_ref[...]
    deg = degp_ref[0, :, 0:1] + degp_ref[1, :, 0:1] + 1.0
    dinv = lax.rsqrt(jnp.maximum(deg, 1.0))
    h1 = jnp.dot(x, we1_ref[...], preferred_element_type=f32)
    dcol = (1.0 - 2.0 * x[:, 0:1]) * dinv
    pad = jnp.zeros((R, 62), f32)
    t1_ref[...] = jnp.concatenate([h1 * dinv, dcol, dinv, pad], axis=1)


def _t1(x, degp, we1):
    return pl.pallas_call(
        _t1_body,
        grid=(G,),
        in_specs=[_row_spec(IN_DIM), _part_spec(W), _full_spec(IN_DIM, HID)],
        out_specs=_row_spec(W),
        out_shape=jax.ShapeDtypeStruct((NPAD, W), f32),
    )(x, degp, we1)


def _t2_body(t1_ref, a1p_ref, we1r0_ref, be1_ref, t2_ref):
    t1 = t1_ref[...]
    dinv = t1[:, 65:66]
    m = dinv * (a1p_ref[0] + a1p_ref[1] + t1)
    pre1 = m[:, :64] + be1_ref[...]
    pre1cf = pre1 + m[:, 64:65] * we1r0_ref[...]
    a1 = jnp.maximum(pre1, 0.0)
    a1cf = jnp.maximum(pre1cf, 0.0)
    t2_ref[...] = jnp.concatenate([a1 * dinv, a1cf * dinv], axis=1)


def _t2(t1, a1p, we1r0, be1):
    return pl.pallas_call(
        _t2_body,
        grid=(G,),
        in_specs=[_row_spec(W), _part_spec(W), _full_spec(1, HID),
                  _full_spec(1, HID)],
        out_specs=_row_spec(W),
        out_shape=jax.ShapeDtypeStruct((NPAD, W), f32),
    )(t1, a1p, we1r0, be1)


def _t3_body(t2_ref, a2p_ref, t1_ref, we2_ref, be2_ref, z_ref, t3_ref):
    dinv = t1_ref[:, 65:66]
    m = dinv * (a2p_ref[0] + a2p_ref[1] + t2_ref[...])
    we2 = we2_ref[...]
    be2 = be2_ref[...]
    z = jnp.dot(m[:, :64], we2, preferred_element_type=f32) + be2
    zcf = jnp.dot(m[:, 64:], we2, preferred_element_type=f32) + be2
    z_ref[...] = z
    pad = jnp.zeros((R, 32), f32)
    t3_ref[...] = jnp.concatenate(
        [z[:, :32], z[:, 32:], zcf[:, :32], pad], axis=1) * dinv


def _t3(t2, a2p, t1, we2, be2):
    return pl.pallas_call(
        _t3_body,
        grid=(G,),
        in_specs=[_row_spec(W), _part_spec(W), _row_spec(W),
                  _full_spec(HID, HID), _full_spec(1, HID)],
        out_specs=[_row_spec(64), _row_spec(W)],
        out_shape=[jax.ShapeDtypeStruct((NPAD, 64), f32),
                   jax.ShapeDtypeStruct((NPAD, W), f32)],
    )(t2, a2p, t1, we2, be2)


def _t4_body(t3_ref, a3p_ref, t1_ref, wa11_ref, wa21_ref, ws_ref, ba11_ref,
             ba21_ref, bs_ref, hs_ref, t4_ref, t5_ref):
    dinv = t1_ref[:, 65:66]
    m = dinv * (a3p_ref[0] + a3p_ref[1] + t3_ref[...])
    m_s, m_ns, m_scf = m[:, :32], m[:, 32:64], m[:, 64:96]
    wa11 = wa11_ref[...]
    ba11 = ba11_ref[...]
    a_s1 = jnp.maximum(
        jnp.dot(m_s, wa11, preferred_element_type=f32) + ba11, 0.0)
    a_ns1 = jnp.maximum(
        jnp.dot(m_ns, wa21_ref[...], preferred_element_type=f32)
        + ba21_ref[...], 0.0)
    a_scf1 = jnp.maximum(
        jnp.dot(m_scf, wa11, preferred_element_type=f32) + ba11, 0.0)
    hs_ref[...] = jnp.dot(m_ns, ws_ref[...],
                          preferred_element_type=f32) + bs_ref[...]
    t4_ref[...] = jnp.concatenate([a_s1, a_ns1], axis=1) * dinv
    t5_ref[...] = jnp.concatenate(
        [a_scf1 * dinv, jnp.zeros((R, 64), f32)], axis=1)


def _t4(t3, a3p, t1, wa11, wa21, ws, ba11, ba21, bs):
    return pl.pallas_call(
        _t4_body,
        grid=(G,),
        in_specs=[_row_spec(W), _part_spec(W), _row_spec(W),
                  _full_spec(32, HID), _full_spec(32, HID),
                  _full_spec(32, HID), _full_spec(1, HID),
                  _full_spec(1, HID), _full_spec(1, HID)],
        out_specs=[_row_spec(64), _row_spec(W), _row_spec(W)],
        out_shape=[jax.ShapeDtypeStruct((NPAD, 64), f32),
                   jax.ShapeDtypeStruct((NPAD, W), f32),
                   jax.ShapeDtypeStruct((NPAD, W), f32)],
    )(t3, a3p, t1, wa11, wa21, ws, ba11, ba21, bs)


def _t5_body(t4_ref, t5_ref, a4p_ref, a5p_ref, t1_ref, wa12_ref, wa22_ref,
             ba12_ref, ba22_ref, xs_ref, xns_ref, xscf_ref):
    dinv = t1_ref[:, 65:66]
    m4 = dinv * (a4p_ref[0] + a4p_ref[1] + t4_ref[...])
    m5 = dinv * (a5p_ref[0] + a5p_ref[1] + t5_ref[...])
    wa12 = wa12_ref[...]
    ba12 = ba12_ref[...]
    xs_ref[...] = jnp.dot(m4[:, :64], wa12,
                          preferred_element_type=f32) + ba12
    xns_ref[...] = jnp.dot(m4[:, 64:], wa22_ref[...],
                           preferred_element_type=f32) + ba22_ref[...]
    xscf_ref[...] = jnp.dot(m5[:, :64], wa12,
                            preferred_element_type=f32) + ba12


def _t5(t4, t5, a4p, a5p, t1, wa12, wa22, ba12, ba22):
    return pl.pallas_call(
        _t5_body,
        grid=(G,),
        in_specs=[_row_spec(W), _row_spec(W), _part_spec(W), _part_spec(W),
                  _row_spec(W), _full_spec(HID, IN_DIM),
                  _full_spec(HID, IN_DIM), _full_spec(1, 128),
                  _full_spec(1, 128)],
        out_specs=[_row_spec(128), _row_spec(128), _row_spec(128)],
        out_shape=[jax.ShapeDtypeStruct((NPAD, 128), f32),
                   jax.ShapeDtypeStruct((NPAD, 128), f32),
                   jax.ShapeDtypeStruct((NPAD, 128), f32)],
    )(t4, t5, a4p, a5p, t1, wa12, wa22, ba12, ba22)


def _t6_body(a_ref, b_ref, out_ref):
    out_ref[...] = lax.dot_general(
        a_ref[...], b_ref[...], (((1,), (1,)), ((), ())),
        preferred_element_type=f32)


def _t6(hs):
    M = 2048
    gm = pl.cdiv(N, M)
    return pl.pallas_call(
        _t6_body,
        grid=(gm, gm),
        in_specs=[pl.BlockSpec((M, HID), lambda i, j: (i, 0)),
                  pl.BlockSpec((M, HID), lambda i, j: (j, 0))],
        out_specs=pl.BlockSpec((M, M), lambda i, j: (i, j)),
        out_shape=jax.ShapeDtypeStruct((N, N), f32),
    )(hs, hs)


def kernel(x, W_e1, b_e1, W_e2, b_e2, Wa11, ba11, Wa12, ba12, Wa21, ba21,
           Wa22, ba22, Ws, bs, edge_index):
    src = edge_index[0].astype(jnp.int32).reshape(NW, PER_W * B)
    dst = edge_index[1].astype(jnp.int32).reshape(NW, PER_W, B)
    xp = jnp.pad(x, ((0, NPAD - N), (0, 0)))
    ones = jnp.ones((B, W), f32)
    z128 = jnp.zeros((RPT, W), f32)

    degp = _deg_pass(ones, src, dst, z128)
    t1 = _t1(xp, degp, W_e1)
    a1p = _pass128(t1, src, dst, z128)
    t2 = _t2(t1, a1p, W_e1[0:1, :], b_e1.reshape(1, -1))
    a2p = _pass128(t2, src, dst, z128)
    z, t3 = _t3(t2, a2p, t1, W_e2, b_e2.reshape(1, -1))
    a3p = _pass128(t3, src, dst, z128)
    hs, t4, t5 = _t4(t3, a3p, t1, Wa11, Wa21, Ws, ba11.reshape(1, -1),
                     ba21.reshape(1, -1), bs.reshape(1, -1))
    a4p = _pass128(t4, src, dst, z128)
    a5p = _pass128(t5, src, dst, z128)
    s_ = _t6(hs)
    xs, xns, xscf = _t5(t4, t5, a4p, a5p, t1, Wa12, Wa22,
                        ba12.reshape(1, -1), ba22.reshape(1, -1))
    return (z[:N, :32], z[:N, 32:], xs[:N], xns[:N], xscf[:N], s_)
